# Initial kernel scaffold; baseline (speedup 1.0000x reference)
import jax
import jax.numpy as jnp
from jax.experimental import pallas as pl


def kernel(x, edge_index, W1, b1, W2, b2, W3, b3):
    raise NotImplementedError("write your pallas kernel here")


# speedup vs baseline: 6.5405x; 6.5405x over previous
import functools

import jax
import jax.numpy as jnp
from jax import lax
from jax.experimental import pallas as pl
from jax.experimental.pallas import tpu as pltpu
from jax.experimental.pallas import tpu_sc as plsc

N = 10000
E = 320000
D_IN = 128
D_HID = 128
N_CLS = 64

NC = 2
NS = 16
ROW_STRIDE = 624
ROWS_PER_TILE = 640
CHUNK = 128
EDGES_PER_TILE = E // NS
NCHUNK = -(-EDGES_PER_TILE // CHUNK)
EPAD = NCHUNK * CHUNK


def _mm_bias_body(x_ref, w_ref, b_ref, z_ref, zb_ref):
    f2 = z_ref.shape[-1]
    z = jnp.dot(x_ref[...], w_ref[...], preferred_element_type=jnp.float32)
    zb = z + b_ref[...]
    z_ref[0] = z[:, :f2]
    z_ref[1] = z[:, f2:]
    zb_ref[0] = zb[:, :f2]
    zb_ref[1] = zb[:, f2:]


def _mm_bias(x, w, b):
    d = w.shape[1]
    out = jax.ShapeDtypeStruct((NC, N, d // NC), jnp.float32)
    return pl.pallas_call(_mm_bias_body, out_shape=(out, out))(x, w, b)


def _mlp_body(m_ref, w2_ref, b2_ref, w3_ref, b3_ref, y_ref, yb_ref):
    f2 = y_ref.shape[-1]
    m = jnp.concatenate([m_ref[0], m_ref[1]], axis=1)
    h = jnp.maximum(m, 0.0)
    h2 = jnp.dot(h, w2_ref[...], preferred_element_type=jnp.float32) + b2_ref[...]
    x1 = jnp.maximum(h2, 0.0)
    y = jnp.dot(x1, w3_ref[...], preferred_element_type=jnp.float32)
    yb = y + b3_ref[...]
    y_ref[0] = y[:, :f2]
    y_ref[1] = y[:, f2:]
    yb_ref[0] = yb[:, :f2]
    yb_ref[1] = yb[:, f2:]


def _mlp(m, w2, b2, w3, b3):
    out = jax.ShapeDtypeStruct((NC, N, N_CLS // NC), jnp.float32)
    return pl.pallas_call(_mlp_body, out_shape=(out, out))(m, w2, b2, w3, b3)


def _make_sc_agg(f2):
    mesh = plsc.VectorSubcoreMesh(core_axis_name="c", subcore_axis_name="s")

    @functools.partial(
        pl.kernel,
        out_type=jax.ShapeDtypeStruct((NC, N, f2), jnp.float32),
        mesh=mesh,
        scratch_types=[
            pltpu.VMEM((NCHUNK, CHUNK), jnp.int32),
            pltpu.VMEM((NCHUNK, CHUNK), jnp.int32),
            pltpu.VMEM((CHUNK, f2), jnp.float32),
            pltpu.VMEM((ROWS_PER_TILE, f2), jnp.float32),
            pltpu.VMEM_SHARED((N + 16, f2), jnp.float32),
            pltpu.SemaphoreType.DMA,
        ],
        compiler_params=pltpu.CompilerParams(use_tc_tiling_on_sc=False),
    )
    def agg(table_hbm, init_hbm, src_hbm, dst_hbm, out_hbm,
            src_v, dst_v, rows_v, stage_v, acc_s, sem):
        c = lax.axis_index("c")
        s = lax.axis_index("s")
        pltpu.sync_copy(src_hbm.at[s], src_v)
        pltpu.sync_copy(dst_hbm.at[s], dst_v)
        r0 = s * ROW_STRIDE
        pltpu.sync_copy(init_hbm.at[c].at[pl.ds(r0, ROWS_PER_TILE)], stage_v)
        pltpu.sync_copy(stage_v, acc_s.at[pl.ds(r0, ROWS_PER_TILE)])
        plsc.subcore_barrier()

        def body(j, carry):
            pltpu.async_copy(table_hbm.at[c].at[src_v.at[j]], rows_v, sem).wait()
            pltpu.sync_copy(rows_v, acc_s.at[dst_v.at[j]], add=True)
            return carry

        lax.fori_loop(0, NCHUNK, body, 0)
        plsc.subcore_barrier()
        pltpu.sync_copy(acc_s.at[pl.ds(r0, ROWS_PER_TILE)], stage_v)
        pltpu.sync_copy(stage_v, out_hbm.at[c].at[pl.ds(r0, ROWS_PER_TILE)])

    return agg


_sc_agg_64 = _make_sc_agg(D_HID // NC)
_sc_agg_32 = _make_sc_agg(N_CLS // NC)


@jax.jit
def kernel(x, edge_index, W1, b1, W2, b2, W3, b3):
    src = edge_index[0].astype(jnp.int32).reshape(NS, EDGES_PER_TILE)
    dst = edge_index[1].astype(jnp.int32).reshape(NS, EDGES_PER_TILE)
    pad = EPAD - EDGES_PER_TILE
    src_p = jnp.pad(src, ((0, 0), (0, pad))).reshape(NS, NCHUNK, CHUNK)
    dst_p = jnp.pad(dst, ((0, 0), (0, pad)), constant_values=N).reshape(
        NS, NCHUNK, CHUNK)

    zs, zbs = _mm_bias(x, W1, b1.reshape(1, D_HID))
    ms = _sc_agg_64(zs, zbs, src_p, dst_p)

    ys, ybs = _mlp(ms, W2, b2.reshape(1, D_HID), W3, b3.reshape(1, N_CLS))
    outs = _sc_agg_32(ys, ybs, src_p, dst_p)

    return outs.transpose(1, 0, 2).reshape(N, N_CLS)
